# Initial kernel scaffold; baseline (speedup 1.0000x reference)
import jax
import jax.numpy as jnp
from jax.experimental import pallas as pl


def kernel(x, Wg, We):
    raise NotImplementedError("write your pallas kernel here")


# speedup vs baseline: 1.9865x; 1.9865x over previous
import jax
import jax.numpy as jnp
from jax.experimental import pallas as pl
from jax.experimental.pallas import tpu as pltpu

_G = 8
_EG = 8
_E = _G * _EG
_K = 2
_BLOCK = 1024


def _router_kernel(x_ref, wg_ref, we_ref, mask_ref, w_ref):
    x = x_ref[...]
    dims = (((1,), (1,)), ((), ()))
    lg = jax.lax.dot_general(x, wg_ref[...], dims,
                             preferred_element_type=jnp.float32)
    le = jax.lax.dot_general(x, we_ref[...], dims,
                             preferred_element_type=jnp.float32)

    gm = jnp.max(lg, axis=-1, keepdims=True)
    gpu = jnp.exp(lg - gm)
    gp = gpu / jnp.sum(gpu, axis=-1, keepdims=True)

    m = jnp.max(le, axis=-1, keepdims=True)
    p0 = jnp.exp(le - m)
    b = x.shape[0]
    s_parts = []
    g_parts = []
    for g in range(_G):
        blk = p0[:, g * _EG:(g + 1) * _EG]
        sg = jnp.sum(blk, axis=-1, keepdims=True)
        s_parts.append(jnp.broadcast_to(sg, (b, _EG)))
        g_parts.append(jnp.broadcast_to(gp[:, g:g + 1], (b, _EG)))
    s = jnp.concatenate(s_parts, axis=-1)
    gpb = jnp.concatenate(g_parts, axis=-1)
    ep = p0 / s

    w = gpb * ep
    vmask = jnp.where((gpb >= (1.0 / _G)) & (ep >= (1.0 / _EG)), 1.0, 0.0)
    nsel = jnp.sum(vmask, axis=-1, keepdims=True)

    lane = jax.lax.broadcasted_iota(jnp.int32, w.shape, 1)
    m1 = jnp.max(w, axis=-1, keepdims=True)
    i1 = jnp.min(jnp.where(w == m1, lane, _E), axis=-1, keepdims=True)
    w2 = jnp.where(lane == i1, -1.0, w)
    m2 = jnp.max(w2, axis=-1, keepdims=True)
    i2 = jnp.min(jnp.where(w2 == m2, lane, _E), axis=-1, keepdims=True)
    tmask = jnp.where((lane == i1) | (lane == i2), 1.0, 0.0)

    fmask = jnp.where(nsel < float(_K), tmask, vmask)
    sw = w * fmask
    ws = jnp.maximum(jnp.sum(sw, axis=-1, keepdims=True), 1e-9)
    w_ref[...] = sw / ws
    mask_ref[...] = fmask


@jax.jit
def kernel(x, Wg, We):
    n, d = x.shape
    mask, w = pl.pallas_call(
        _router_kernel,
        grid=(n // _BLOCK,),
        in_specs=[
            pl.BlockSpec((_BLOCK, d), lambda i: (i, 0)),
            pl.BlockSpec((_G, d), lambda i: (0, 0)),
            pl.BlockSpec((_E, d), lambda i: (0, 0)),
        ],
        out_specs=[
            pl.BlockSpec((_BLOCK, _E), lambda i: (i, 0)),
            pl.BlockSpec((_BLOCK, _E), lambda i: (i, 0)),
        ],
        out_shape=[
            jax.ShapeDtypeStruct((n, _E), jnp.float32),
            jax.ShapeDtypeStruct((n, _E), jnp.float32),
        ],
    )(x, Wg, We)
    return mask.astype(jnp.bool_), w


# speedup vs baseline: 6.7628x; 3.4044x over previous
import jax
import jax.numpy as jnp
from jax.experimental import pallas as pl
from jax.experimental.pallas import tpu as pltpu

_G = 8
_EG = 8
_E = _G * _EG
_K = 2
_BLOCK = 1024


def _router_kernel(wc_ref, x_ref, mask_ref, w_ref):
    x = x_ref[...]
    lt = jax.lax.dot_general(wc_ref[...], x, (((1,), (1,)), ((), ())),
                             preferred_element_type=jnp.float32)
    lg = lt[0:_G, :]
    le = lt[_G:_G + _E, :]
    b = x.shape[0]

    gm = jnp.max(lg, axis=0, keepdims=True)
    gpu = jnp.exp(lg - gm)
    gp = gpu / jnp.sum(gpu, axis=0, keepdims=True)

    m = jnp.max(le, axis=0, keepdims=True)
    p0 = jnp.exp(le - m)
    s_parts = []
    g_parts = []
    for g in range(_G):
        blk = p0[g * _EG:(g + 1) * _EG, :]
        sg = jnp.sum(blk, axis=0, keepdims=True)
        s_parts.append(jnp.broadcast_to(sg, (_EG, b)))
        g_parts.append(jnp.broadcast_to(gp[g:g + 1, :], (_EG, b)))
    s = jnp.concatenate(s_parts, axis=0)
    gpb = jnp.concatenate(g_parts, axis=0)
    ep = p0 / s

    w = gpb * ep
    vmask = jnp.where((gpb >= (1.0 / _G)) & (ep >= (1.0 / _EG)), 1.0, 0.0)
    nsel = jnp.sum(vmask, axis=0, keepdims=True)

    sub = jax.lax.broadcasted_iota(jnp.int32, w.shape, 0)
    m1 = jnp.max(w, axis=0, keepdims=True)
    i1 = jnp.min(jnp.where(w == m1, sub, _E), axis=0, keepdims=True)
    w2 = jnp.where(sub == i1, -1.0, w)
    m2 = jnp.max(w2, axis=0, keepdims=True)
    i2 = jnp.min(jnp.where(w2 == m2, sub, _E), axis=0, keepdims=True)
    tmask = jnp.where((sub == i1) | (sub == i2), 1.0, 0.0)

    fmask = jnp.where(nsel < float(_K), tmask, vmask)
    sw = w * fmask
    ws = jnp.maximum(jnp.sum(sw, axis=0, keepdims=True), 1e-9)
    w_ref[...] = (sw / ws).T
    mask_ref[...] = fmask.T


@jax.jit
def kernel(x, Wg, We):
    n, d = x.shape
    wc = jnp.concatenate([Wg, We], axis=0)
    mask, w = pl.pallas_call(
        _router_kernel,
        grid=(n // _BLOCK,),
        in_specs=[
            pl.BlockSpec((_G + _E, d), lambda i: (0, 0)),
            pl.BlockSpec((_BLOCK, d), lambda i: (i, 0)),
        ],
        out_specs=[
            pl.BlockSpec((_BLOCK, _E), lambda i: (i, 0)),
            pl.BlockSpec((_BLOCK, _E), lambda i: (i, 0)),
        ],
        out_shape=[
            jax.ShapeDtypeStruct((n, _E), jnp.float32),
            jax.ShapeDtypeStruct((n, _E), jnp.float32),
        ],
    )(wc, x)
    return mask.astype(jnp.bool_), w


# speedup vs baseline: 7.6640x; 1.1333x over previous
import jax
import jax.numpy as jnp
from jax.experimental import pallas as pl
from jax.experimental.pallas import tpu as pltpu

_G = 8
_EG = 8
_E = _G * _EG
_K = 2
_BLOCK = 2048


def _router_kernel(wc_ref, x_ref, mask_ref, w_ref):
    x = x_ref[...]
    lt = jax.lax.dot_general(wc_ref[...], x, (((1,), (1,)), ((), ())),
                             preferred_element_type=jnp.float32)
    lg = lt[0:_G, :]
    le = lt[_G:_G + _E, :]
    b = x.shape[0]

    gm = jnp.max(lg, axis=0, keepdims=True)
    gpu = jnp.exp(lg - gm)
    gp = gpu / jnp.sum(gpu, axis=0, keepdims=True)

    m = jnp.max(le, axis=0, keepdims=True)
    p0 = jnp.exp(le - m)
    s_parts = []
    g_parts = []
    for g in range(_G):
        blk = p0[g * _EG:(g + 1) * _EG, :]
        sg = jnp.sum(blk, axis=0, keepdims=True)
        s_parts.append(jnp.broadcast_to(sg, (_EG, b)))
        g_parts.append(jnp.broadcast_to(gp[g:g + 1, :], (_EG, b)))
    s = jnp.concatenate(s_parts, axis=0)
    gpb = jnp.concatenate(g_parts, axis=0)
    ep = p0 / s

    w = gpb * ep
    vmask = jnp.where((gpb >= (1.0 / _G)) & (ep >= (1.0 / _EG)), 1.0, 0.0)
    nsel = jnp.sum(vmask, axis=0, keepdims=True)

    sub = jax.lax.broadcasted_iota(jnp.int32, w.shape, 0)
    m1 = jnp.max(w, axis=0, keepdims=True)
    i1 = jnp.min(jnp.where(w == m1, sub, _E), axis=0, keepdims=True)
    w2 = jnp.where(sub == i1, -1.0, w)
    m2 = jnp.max(w2, axis=0, keepdims=True)
    i2 = jnp.min(jnp.where(w2 == m2, sub, _E), axis=0, keepdims=True)
    tmask = jnp.where((sub == i1) | (sub == i2), 1.0, 0.0)

    fmask = jnp.where(nsel < float(_K), tmask, vmask)
    sw = w * fmask
    ws = jnp.maximum(jnp.sum(sw, axis=0, keepdims=True), 1e-9)
    w_ref[...] = (sw / ws).T
    mask_ref[...] = fmask.T


@jax.jit
def kernel(x, Wg, We):
    n, d = x.shape
    wc = jnp.concatenate([Wg, We], axis=0)
    mask, w = pl.pallas_call(
        _router_kernel,
        grid=(n // _BLOCK,),
        in_specs=[
            pl.BlockSpec((_G + _E, d), lambda i: (0, 0)),
            pl.BlockSpec((_BLOCK, d), lambda i: (i, 0)),
        ],
        out_specs=[
            pl.BlockSpec((_BLOCK, _E), lambda i: (i, 0)),
            pl.BlockSpec((_BLOCK, _E), lambda i: (i, 0)),
        ],
        out_shape=[
            jax.ShapeDtypeStruct((n, _E), jnp.float32),
            jax.ShapeDtypeStruct((n, _E), jnp.float32),
        ],
    )(wc, x)
    return mask.astype(jnp.bool_), w


# speedup vs baseline: 7.9135x; 1.0326x over previous
import jax
import jax.numpy as jnp
from jax.experimental import pallas as pl
from jax.experimental.pallas import tpu as pltpu

_G = 8
_EG = 8
_E = _G * _EG
_K = 2
_BLOCK = 4096


def _router_kernel(wc_ref, x_ref, mask_ref, w_ref):
    x = x_ref[...]
    lt = jax.lax.dot_general(wc_ref[...], x, (((1,), (1,)), ((), ())),
                             preferred_element_type=jnp.float32)
    lg = lt[0:_G, :]
    le = lt[_G:_G + _E, :]
    b = x.shape[0]

    gm = jnp.max(lg, axis=0, keepdims=True)
    gpu = jnp.exp(lg - gm)
    gp = gpu / jnp.sum(gpu, axis=0, keepdims=True)

    m = jnp.max(le, axis=0, keepdims=True)
    p0 = jnp.exp(le - m)
    s_parts = []
    g_parts = []
    for g in range(_G):
        blk = p0[g * _EG:(g + 1) * _EG, :]
        sg = jnp.sum(blk, axis=0, keepdims=True)
        s_parts.append(jnp.broadcast_to(sg, (_EG, b)))
        g_parts.append(jnp.broadcast_to(gp[g:g + 1, :], (_EG, b)))
    s = jnp.concatenate(s_parts, axis=0)
    gpb = jnp.concatenate(g_parts, axis=0)
    ep = p0 / s

    w = gpb * ep
    vmask = jnp.where((gpb >= (1.0 / _G)) & (ep >= (1.0 / _EG)), 1.0, 0.0)
    nsel = jnp.sum(vmask, axis=0, keepdims=True)

    sub = jax.lax.broadcasted_iota(jnp.int32, w.shape, 0)
    m1 = jnp.max(w, axis=0, keepdims=True)
    i1 = jnp.min(jnp.where(w == m1, sub, _E), axis=0, keepdims=True)
    w2 = jnp.where(sub == i1, -1.0, w)
    m2 = jnp.max(w2, axis=0, keepdims=True)
    i2 = jnp.min(jnp.where(w2 == m2, sub, _E), axis=0, keepdims=True)
    tmask = jnp.where((sub == i1) | (sub == i2), 1.0, 0.0)

    fmask = jnp.where(nsel < float(_K), tmask, vmask)
    sw = w * fmask
    ws = jnp.maximum(jnp.sum(sw, axis=0, keepdims=True), 1e-9)
    w_ref[...] = (sw / ws).T
    mask_ref[...] = fmask.T


@jax.jit
def kernel(x, Wg, We):
    n, d = x.shape
    wc = jnp.concatenate([Wg, We], axis=0)
    mask, w = pl.pallas_call(
        _router_kernel,
        grid=(n // _BLOCK,),
        in_specs=[
            pl.BlockSpec((_G + _E, d), lambda i: (0, 0)),
            pl.BlockSpec((_BLOCK, d), lambda i: (i, 0)),
        ],
        out_specs=[
            pl.BlockSpec((_BLOCK, _E), lambda i: (i, 0)),
            pl.BlockSpec((_BLOCK, _E), lambda i: (i, 0)),
        ],
        out_shape=[
            jax.ShapeDtypeStruct((n, _E), jnp.float32),
            jax.ShapeDtypeStruct((n, _E), jnp.float32),
        ],
    )(wc, x)
    return mask.astype(jnp.bool_), w


# speedup vs baseline: 7.9329x; 1.0025x over previous
import jax
import jax.numpy as jnp
from jax.experimental import pallas as pl
from jax.experimental.pallas import tpu as pltpu

_G = 8
_EG = 8
_E = _G * _EG
_K = 2
_BLOCK = 4096


def _router_kernel(wc_ref, x_ref, mask_ref, w_ref):
    x = x_ref[...]
    lt = jax.lax.dot_general(wc_ref[...], x, (((1,), (1,)), ((), ())),
                             preferred_element_type=jnp.float32)
    lg = lt[0:_G, :]
    le = lt[_G:_G + _E, :]
    b = x.shape[0]

    gm = jnp.max(lg, axis=0, keepdims=True)
    gpu = jnp.exp(lg - gm)
    gp = gpu / jnp.sum(gpu, axis=0, keepdims=True)

    m = jnp.max(le, axis=0, keepdims=True)
    p0 = jnp.exp(le - m)
    s_parts = []
    g_parts = []
    for g in range(_G):
        blk = p0[g * _EG:(g + 1) * _EG, :]
        sg = jnp.sum(blk, axis=0, keepdims=True)
        s_parts.append(jnp.broadcast_to(sg, (_EG, b)))
        g_parts.append(jnp.broadcast_to(gp[g:g + 1, :], (_EG, b)))
    s = jnp.concatenate(s_parts, axis=0)
    gpb = jnp.concatenate(g_parts, axis=0)
    ep = p0 / s

    w = gpb * ep
    vmask = jnp.where((gpb >= (1.0 / _G)) & (ep >= (1.0 / _EG)), 1.0, 0.0)
    nsel = jnp.sum(vmask, axis=0, keepdims=True)

    sub = jax.lax.broadcasted_iota(jnp.int32, w.shape, 0)
    m1 = jnp.max(w, axis=0, keepdims=True)
    i1 = jnp.min(jnp.where(w == m1, sub, _E), axis=0, keepdims=True)
    w2 = jnp.where(sub == i1, -1.0, w)
    m2 = jnp.max(w2, axis=0, keepdims=True)
    i2 = jnp.min(jnp.where(w2 == m2, sub, _E), axis=0, keepdims=True)
    tmask = jnp.where((sub == i1) | (sub == i2), 1.0, 0.0)

    fmask = jnp.where(nsel < float(_K), tmask, vmask)
    sw = w * fmask
    ws = jnp.maximum(jnp.sum(sw, axis=0, keepdims=True), 1e-9)
    w_ref[...] = (sw / ws).T
    mask_ref[...] = fmask.T


@jax.jit
def kernel(x, Wg, We):
    n, d = x.shape
    wc = jnp.concatenate([Wg, We], axis=0)
    mask, w = pl.pallas_call(
        _router_kernel,
        grid=(n // _BLOCK,),
        in_specs=[
            pl.BlockSpec((_G + _E, d), lambda i: (0, 0)),
            pl.BlockSpec((_BLOCK, d), lambda i: (i, 0)),
        ],
        out_specs=[
            pl.BlockSpec((_BLOCK, _E), lambda i: (i, 0)),
            pl.BlockSpec((_BLOCK, _E), lambda i: (i, 0)),
        ],
        out_shape=[
            jax.ShapeDtypeStruct((n, _E), jnp.float32),
            jax.ShapeDtypeStruct((n, _E), jnp.float32),
        ],
        compiler_params=pltpu.CompilerParams(
            dimension_semantics=("parallel",)),
    )(wc, x)
    return mask.astype(jnp.bool_), w


# speedup vs baseline: 8.5180x; 1.0738x over previous
import jax
import jax.numpy as jnp
from jax.experimental import pallas as pl
from jax.experimental.pallas import tpu as pltpu

_G = 8
_EG = 8
_E = _G * _EG
_K = 2
_BLOCK = 4096


def _router_kernel(wc_ref, x_ref, mask_ref, w_ref):
    x = x_ref[...]
    lt = jax.lax.dot_general(wc_ref[...], x, (((1,), (1,)), ((), ())),
                             preferred_element_type=jnp.float32)
    lg = lt[0:_G, :]
    le = lt[_G:_G + _E, :]
    b = x.shape[0]

    gm = jnp.max(lg, axis=0, keepdims=True)
    gpu = jnp.exp(lg - gm)
    gp = gpu / jnp.sum(gpu, axis=0, keepdims=True)

    m = jnp.max(le, axis=0, keepdims=True)
    p0 = jnp.exp(le - m)
    s_parts = []
    g_parts = []
    for g in range(_G):
        blk = p0[g * _EG:(g + 1) * _EG, :]
        sg = jnp.sum(blk, axis=0, keepdims=True)
        s_parts.append(jnp.broadcast_to(sg, (_EG, b)))
        g_parts.append(jnp.broadcast_to(gp[g:g + 1, :], (_EG, b)))
    s = jnp.concatenate(s_parts, axis=0)
    gpb = jnp.concatenate(g_parts, axis=0)
    ep = p0 / s

    w = gpb * ep
    vmask = jnp.where((gpb >= (1.0 / _G)) & (ep >= (1.0 / _EG)), 1.0, 0.0)
    nsel = jnp.sum(vmask, axis=0, keepdims=True)

    sub = jax.lax.broadcasted_iota(jnp.int32, w.shape, 0)
    m1 = jnp.max(w, axis=0, keepdims=True)
    i1 = jnp.min(jnp.where(w == m1, sub, _E), axis=0, keepdims=True)
    w2 = jnp.where(sub == i1, -1.0, w)
    m2 = jnp.max(w2, axis=0, keepdims=True)
    i2 = jnp.min(jnp.where(w2 == m2, sub, _E), axis=0, keepdims=True)
    tmask = jnp.where((sub == i1) | (sub == i2), 1.0, 0.0)

    fmask = jnp.where(nsel < float(_K), tmask, vmask)
    sw = w * fmask
    ws = jnp.maximum(jnp.sum(sw, axis=0, keepdims=True), 1e-9)
    w_ref[...] = (sw / ws).T
    mask_ref[...] = fmask.T.astype(jnp.int8)


@jax.jit
def kernel(x, Wg, We):
    n, d = x.shape
    wc = jnp.concatenate([Wg, We], axis=0)
    mask, w = pl.pallas_call(
        _router_kernel,
        grid=(n // _BLOCK,),
        in_specs=[
            pl.BlockSpec((_G + _E, d), lambda i: (0, 0)),
            pl.BlockSpec((_BLOCK, d), lambda i: (i, 0)),
        ],
        out_specs=[
            pl.BlockSpec((_BLOCK, _E), lambda i: (i, 0)),
            pl.BlockSpec((_BLOCK, _E), lambda i: (i, 0)),
        ],
        out_shape=[
            jax.ShapeDtypeStruct((n, _E), jnp.int8),
            jax.ShapeDtypeStruct((n, _E), jnp.float32),
        ],
        compiler_params=pltpu.CompilerParams(
            dimension_semantics=("parallel",)),
    )(wc, x)
    return mask.astype(jnp.bool_), w
